# baseline (speedup 1.0000x reference)
import jax
import jax.numpy as jnp
from jax.experimental import pallas as pl
from jax.experimental.pallas import tpu as pltpu

HID = 512
ATTN = 384
VOCAB = 30
B = 8
L = 16
N = 196


def _step(fs_ref, ms_ref, h0h_ref, h0c_ref, wat_ref, wht_ref, vt_ref,
          wih_ref, whh_ref, b2_ref, wlt_ref, blt_ref,
          betas_ref, logits_ref, probs_ref, hout_ref, cout_ref,
          h_s, c_s):
    i = pl.program_id(0)

    @pl.when(i == 0)
    def _():
        h_s[:] = h0h_ref[0]
        c_s[:] = h0c_ref[0]

    prev = h_s[:]
    hw = jnp.dot(wht_ref[:], prev, preferred_element_type=jnp.float32)

    aws = []
    for b in range(B):
        fi_b = fs_ref[b, 0]
        fw_b = jnp.dot(wat_ref[:], fi_b, preferred_element_type=jnp.float32)
        t_b = jnp.tanh(fw_b + hw[:, b:b + 1])
        aws.append(jnp.dot(vt_ref[:], t_b, preferred_element_type=jnp.float32))
    aw = jnp.concatenate(aws, axis=0)

    m = jnp.max(aw, axis=1, keepdims=True)
    e = jnp.exp(aw - m)
    sm = e / jnp.sum(e, axis=1, keepdims=True)
    awm = sm * ms_ref[0]
    betas_ref[0] = awm

    denom = jnp.clip(jnp.sum(awm, axis=1, keepdims=True), 1e-5, None)
    awn = awm / denom
    scols = []
    for b in range(B):
        fi_b = fs_ref[b, 0]
        scols.append(jnp.dot(fi_b, awn[b:b + 1, :].T,
                             preferred_element_type=jnp.float32))
    s = jnp.concatenate(scols, axis=1)

    gates = (jnp.dot(wih_ref[:], s, preferred_element_type=jnp.float32)
             + jnp.dot(whh_ref[:], prev, preferred_element_type=jnp.float32)
             + b2_ref[:])
    ig = jax.nn.sigmoid(gates[:HID])
    fg = jax.nn.sigmoid(gates[HID:2 * HID])
    gg = jnp.tanh(gates[2 * HID:3 * HID])
    og = jax.nn.sigmoid(gates[3 * HID:])
    c = fg * c_s[:] + ig * gg
    h = og * jnp.tanh(c)
    h_s[:] = h
    c_s[:] = c
    hout_ref[0] = h
    cout_ref[0] = c

    lg = jnp.dot(wlt_ref[:], h, preferred_element_type=jnp.float32) + blt_ref[:]
    logits_ref[0] = lg
    pm = jnp.max(lg, axis=0, keepdims=True)
    pe = jnp.exp(lg - pm)
    probs_ref[0] = pe / jnp.sum(pe, axis=0, keepdims=True)


def kernel(Fs, h0_h, h0_c, Ms, Wa, Wh, v, W_ih, b_ih, W_hh, b_hh, W_lt, b_lt):
    B_, L_, Fd, hm, wm = Fs.shape
    Fsn = Fs.reshape(B_, L_, Fd, N)
    Msr = jnp.transpose(Ms.reshape(B_, L_, N), (1, 0, 2))
    h0h = jnp.transpose(h0_h, (1, 2, 0))
    h0c = jnp.transpose(h0_c, (1, 2, 0))
    wat = Wa.T
    wht = Wh.T
    vt = v.T
    b2 = (b_ih + b_hh)[:, None]
    blt = b_lt[:, None]

    def full(a):
        nd = a.ndim
        return pl.BlockSpec(a.shape, lambda i, _n=nd: (0,) * _n)

    grid = (L_,)
    out_shapes = (
        jax.ShapeDtypeStruct((L_, B_, N), jnp.float32),
        jax.ShapeDtypeStruct((L_, VOCAB, B_), jnp.float32),
        jax.ShapeDtypeStruct((L_, VOCAB, B_), jnp.float32),
        jax.ShapeDtypeStruct((1, HID, B_), jnp.float32),
        jax.ShapeDtypeStruct((1, HID, B_), jnp.float32),
    )
    betas, logits, probs, hx, cx = pl.pallas_call(
        _step,
        grid=grid,
        in_specs=[
            pl.BlockSpec((B_, 1, Fd, N), lambda i: (0, i, 0, 0)),
            pl.BlockSpec((1, B_, N), lambda i: (i, 0, 0)),
            full(h0h), full(h0c), full(wat), full(wht), full(vt),
            full(W_ih), full(W_hh), full(b2), full(W_lt), full(blt),
        ],
        out_specs=(
            pl.BlockSpec((1, B_, N), lambda i: (i, 0, 0)),
            pl.BlockSpec((1, VOCAB, B_), lambda i: (i, 0, 0)),
            pl.BlockSpec((1, VOCAB, B_), lambda i: (i, 0, 0)),
            pl.BlockSpec((1, HID, B_), lambda i: (0, 0, 0)),
            pl.BlockSpec((1, HID, B_), lambda i: (0, 0, 0)),
        ),
        out_shape=out_shapes,
        scratch_shapes=[
            pltpu.VMEM((HID, B_), jnp.float32),
            pltpu.VMEM((HID, B_), jnp.float32),
        ],
        compiler_params=pltpu.CompilerParams(
            dimension_semantics=("arbitrary",),
        ),
    )(Fsn, Msr, h0h, h0c, wat, wht, vt, W_ih, W_hh, b2, W_lt, blt)

    logits_o = jnp.transpose(logits, (2, 0, 1))
    probs_o = jnp.transpose(probs, (2, 0, 1))
    betas_o = jnp.transpose(betas, (1, 0, 2)).reshape(B_, L_, hm, wm)
    hx_o = jnp.transpose(hx, (0, 2, 1))
    cx_o = jnp.transpose(cx, (0, 2, 1))
    return logits_o, probs_o, hx_o, cx_o, betas_o


# speedup vs baseline: 1.3041x; 1.3041x over previous
import jax
import jax.numpy as jnp
from jax.experimental import pallas as pl
from jax.experimental.pallas import tpu as pltpu

HID = 512
ATTN = 384
VOCAB = 30
B = 8
L = 16
N = 196


def _step(fs_ref, ms_ref, h0h_ref, h0c_ref, wa_ref, wh_ref, vt_ref,
          wcat_ref, b2_ref, wlt_ref, blt_ref,
          betas_ref, logits_ref, probs_ref, hout_ref, cout_ref,
          h_s, c_s):
    i = pl.program_id(0)

    @pl.when(i == 0)
    def _():
        h_s[:] = h0h_ref[0]
        c_s[:] = h0c_ref[0]

    fi = fs_ref[0]
    prev = h_s[:]

    fw = jnp.dot(fi, wa_ref[:], preferred_element_type=jnp.float32)
    hw = jnp.dot(prev, wh_ref[:], preferred_element_type=jnp.float32)
    t = jnp.tanh(fw.reshape(N, B, HID) + hw[None, :, :])
    aw = jnp.sum(t * vt_ref[:][None], axis=2)

    awt = jnp.transpose(aw)
    m = jnp.max(awt, axis=1, keepdims=True)
    e = jnp.exp(awt - m)
    sm = e / jnp.sum(e, axis=1, keepdims=True)
    awm = sm * ms_ref[0]
    betas_ref[0] = awm

    denom = jnp.clip(jnp.sum(awm, axis=1, keepdims=True), 1e-5, None)
    awn = jnp.transpose(awm / denom)
    s = jnp.sum(awn[:, :, None] * fi.reshape(N, B, ATTN).astype(jnp.float32),
                axis=0)

    x = jnp.concatenate([s, prev], axis=1)
    gates = jnp.dot(x, wcat_ref[:], preferred_element_type=jnp.float32) + b2_ref[:]
    ig = jax.nn.sigmoid(gates[:, :HID])
    fg = jax.nn.sigmoid(gates[:, HID:2 * HID])
    gg = jnp.tanh(gates[:, 2 * HID:3 * HID])
    og = jax.nn.sigmoid(gates[:, 3 * HID:])
    c = fg * c_s[:] + ig * gg
    h = og * jnp.tanh(c)
    h_s[:] = h
    c_s[:] = c
    hout_ref[0] = h
    cout_ref[0] = c

    lg = jnp.dot(h, wlt_ref[:], preferred_element_type=jnp.float32) + blt_ref[:]
    logits_ref[0] = lg
    pm = jnp.max(lg, axis=1, keepdims=True)
    pe = jnp.exp(lg - pm)
    probs_ref[0] = pe / jnp.sum(pe, axis=1, keepdims=True)


def kernel(Fs, h0_h, h0_c, Ms, Wa, Wh, v, W_ih, b_ih, W_hh, b_hh, W_lt, b_lt):
    B_, L_, Fd, hm, wm = Fs.shape
    Fst = jnp.transpose(Fs.reshape(B_, L_, Fd, N), (1, 3, 0, 2)) \
             .reshape(L_, N * B_, Fd).astype(jnp.bfloat16)
    Msr = jnp.transpose(Ms.reshape(B_, L_, N), (1, 0, 2))
    h0h = jnp.transpose(h0_h, (1, 0, 2))
    h0c = jnp.transpose(h0_c, (1, 0, 2))
    wab = Wa.astype(jnp.bfloat16)
    vt = v.T
    wcat = jnp.concatenate([W_ih.T, W_hh.T], axis=0)
    b2 = (b_ih + b_hh)[None, :]
    wlt = W_lt.T
    blt = b_lt[None, :]

    def full(a):
        nd = a.ndim
        return pl.BlockSpec(a.shape, lambda i, _n=nd: (0,) * _n)

    grid = (L_,)
    out_shapes = (
        jax.ShapeDtypeStruct((L_, B_, N), jnp.float32),
        jax.ShapeDtypeStruct((L_, B_, VOCAB), jnp.float32),
        jax.ShapeDtypeStruct((L_, B_, VOCAB), jnp.float32),
        jax.ShapeDtypeStruct((1, B_, HID), jnp.float32),
        jax.ShapeDtypeStruct((1, B_, HID), jnp.float32),
    )
    betas, logits, probs, hx, cx = pl.pallas_call(
        _step,
        grid=grid,
        in_specs=[
            pl.BlockSpec((1, N * B_, Fd), lambda i: (i, 0, 0)),
            pl.BlockSpec((1, B_, N), lambda i: (i, 0, 0)),
            full(h0h), full(h0c), full(wab), full(Wh), full(vt),
            full(wcat), full(b2), full(wlt), full(blt),
        ],
        out_specs=(
            pl.BlockSpec((1, B_, N), lambda i: (i, 0, 0)),
            pl.BlockSpec((1, B_, VOCAB), lambda i: (i, 0, 0)),
            pl.BlockSpec((1, B_, VOCAB), lambda i: (i, 0, 0)),
            pl.BlockSpec((1, B_, HID), lambda i: (0, 0, 0)),
            pl.BlockSpec((1, B_, HID), lambda i: (0, 0, 0)),
        ),
        out_shape=out_shapes,
        scratch_shapes=[
            pltpu.VMEM((B_, HID), jnp.float32),
            pltpu.VMEM((B_, HID), jnp.float32),
        ],
        compiler_params=pltpu.CompilerParams(
            dimension_semantics=("arbitrary",),
        ),
    )(Fst, Msr, h0h, h0c, wab, Wh, vt, wcat, b2, wlt, blt)

    logits_o = jnp.transpose(logits, (1, 0, 2))
    probs_o = jnp.transpose(probs, (1, 0, 2))
    betas_o = jnp.transpose(betas, (1, 0, 2)).reshape(B_, L_, hm, wm)
    return logits_o, probs_o, hx, cx, betas_o
